# baseline (device time: 158443 ns/iter reference)
import jax
import jax.numpy as jnp
from jax import lax
from jax.experimental import pallas as pl
from jax.experimental.pallas import tpu as pltpu

N_DEV = 4
M = 8192
N = 1024
HALF = M // 2
CHUNK = HALF // N_DEV
SUBS = 2
SUB = CHUNK // SUBS
HOPS = N_DEV - 1


def kernel(x):
    def body(x_ref, out_ref, stage, send0, own, rs_recv_a, rs_recv_b,
             ag_recv_a, ag_recv_b, stage_sems, store_sems,
             rs_send_sems, rs_recv_sems, ag_send_sems, ag_recv_sems):
        my = lax.axis_index("i")
        left = lax.rem(my - 1 + N_DEV, N_DEV)
        right = lax.rem(my + 1, N_DEV)

        def rows(d, c, s):
            return pl.ds(d * HALF + c * CHUNK + s * SUB, SUB)

        def peer(d):
            return right if d == 0 else left

        def rs_chunk(d, h):
            if d == 0:
                return lax.rem(my - h - 1 + N_DEV, N_DEV)
            return lax.rem(my + h + 1, N_DEV)

        def ag_chunk(d, h):
            if d == 0:
                return lax.rem(my - h + N_DEV, N_DEV)
            return lax.rem(my + h, N_DEV)

        rs_recv = (rs_recv_a, rs_recv_b)
        ag_recv = (ag_recv_a, ag_recv_b)

        def fetch(d, c, slot):
            return pltpu.make_async_copy(
                x_ref.at[pl.ds(d * HALF + c * CHUNK, CHUNK), :],
                stage.at[d, slot],
                stage_sems.at[d, slot],
            )

        def store(d, h, s):
            if h == HOPS:
                src = own.at[d, pl.ds(s * SUB, SUB), :]
                c = rs_chunk(d, HOPS - 1)
            else:
                src = ag_recv[d].at[h, pl.ds(s * SUB, SUB), :]
                c = ag_chunk(d, h)
            return pltpu.make_async_copy(
                src, out_ref.at[rows(d, c, s), :], store_sems.at[d, h],
            )

        def rs_rdma(d, h, s):
            if h == 0:
                src = send0.at[d, pl.ds(s * SUB, SUB), :]
            else:
                src = rs_recv[d].at[h - 1, pl.ds(s * SUB, SUB), :]
            return pltpu.make_async_remote_copy(
                src_ref=src,
                dst_ref=rs_recv[d].at[h, pl.ds(s * SUB, SUB), :],
                send_sem=rs_send_sems.at[d, h, s],
                recv_sem=rs_recv_sems.at[d, h, s],
                device_id=(peer(d),),
                device_id_type=pl.DeviceIdType.MESH,
            )

        def ag_rdma(d, h, s):
            if h == 0:
                src = own.at[d, pl.ds(s * SUB, SUB), :]
            else:
                src = ag_recv[d].at[h - 1, pl.ds(s * SUB, SUB), :]
            return pltpu.make_async_remote_copy(
                src_ref=src,
                dst_ref=ag_recv[d].at[h, pl.ds(s * SUB, SUB), :],
                send_sem=ag_send_sems.at[d, h, s],
                recv_sem=ag_recv_sems.at[d, h, s],
                device_id=(peer(d),),
                device_id_type=pl.DeviceIdType.MESH,
            )

        for d in range(2):
            fetch(d, my, 0).start()
            fetch(d, rs_chunk(d, 0), 1).start()

        barrier_sem = pltpu.get_barrier_semaphore()
        for nbr in (left, right):
            pl.semaphore_signal(
                barrier_sem, inc=1,
                device_id=(nbr,), device_id_type=pl.DeviceIdType.MESH,
            )
        pl.semaphore_wait(barrier_sem, 2)

        for d in range(2):
            fetch(d, my, 0).wait()
            send0[d, :, :] = stage[d, 0].astype(jnp.bfloat16)
            for s in range(SUBS):
                rs_rdma(d, 0, s).start()
            fetch(d, rs_chunk(d, 1), 0).start()

        for h in range(1, HOPS):
            slot = h % 2
            for d in range(2):
                fetch(d, rs_chunk(d, h - 1), slot).wait()
            for s in range(SUBS):
                for d in range(2):
                    rs_rdma(d, h - 1, s).wait_recv()
                    sl = pl.ds(s * SUB, SUB)
                    rs_recv[d][h - 1, sl, :] = (
                        rs_recv[d][h - 1, sl, :]
                        + stage[d, slot, sl, :].astype(jnp.bfloat16)
                    )
                    rs_rdma(d, h, s).start()
            if h == 1:
                for d in range(2):
                    fetch(d, rs_chunk(d, 2), 1).start()

        for d in range(2):
            fetch(d, rs_chunk(d, HOPS - 1), 1).wait()
        for s in range(SUBS):
            for d in range(2):
                rs_rdma(d, HOPS - 1, s).wait_recv()
                sl = pl.ds(s * SUB, SUB)
                own[d, sl, :] = (
                    rs_recv[d][HOPS - 1, sl, :]
                    + stage[d, 1, sl, :].astype(jnp.bfloat16)
                )
                ag_rdma(d, 0, s).start()
                store(d, HOPS, s).start()

        for h in range(1, HOPS):
            for s in range(SUBS):
                for d in range(2):
                    ag_rdma(d, h - 1, s).wait_recv()
                    ag_rdma(d, h, s).start()
                    store(d, h - 1, s).start()

        for s in range(SUBS):
            for d in range(2):
                ag_rdma(d, HOPS - 1, s).wait_recv()
                store(d, HOPS - 1, s).start()
        for h in range(HOPS + 1):
            for s in range(SUBS):
                for d in range(2):
                    store(d, h, s).wait()
        for h in range(HOPS):
            for s in range(SUBS):
                for d in range(2):
                    rs_rdma(d, h, s).wait_send()
                    ag_rdma(d, h, s).wait_send()

    return pl.pallas_call(
        body,
        out_shape=jax.ShapeDtypeStruct((M, N), jnp.bfloat16),
        in_specs=[pl.BlockSpec(memory_space=pl.ANY)],
        out_specs=pl.BlockSpec(memory_space=pl.ANY),
        scratch_shapes=[
            pltpu.VMEM((2, 2, CHUNK, N), jnp.float32),
            pltpu.VMEM((2, CHUNK, N), jnp.bfloat16),
            pltpu.VMEM((2, CHUNK, N), jnp.bfloat16),
            pltpu.VMEM((HOPS, CHUNK, N), jnp.bfloat16),
            pltpu.VMEM((HOPS, CHUNK, N), jnp.bfloat16),
            pltpu.VMEM((HOPS, CHUNK, N), jnp.bfloat16),
            pltpu.VMEM((HOPS, CHUNK, N), jnp.bfloat16),
            pltpu.SemaphoreType.DMA((2, 2)),
            pltpu.SemaphoreType.DMA((2, HOPS + 1)),
            pltpu.SemaphoreType.DMA((2, HOPS, SUBS)),
            pltpu.SemaphoreType.DMA((2, HOPS, SUBS)),
            pltpu.SemaphoreType.DMA((2, HOPS, SUBS)),
            pltpu.SemaphoreType.DMA((2, HOPS, SUBS)),
        ],
        compiler_params=pltpu.CompilerParams(
            collective_id=0,
            vmem_limit_bytes=56 * 1024 * 1024,
        ),
    )(x)


# device time: 149710 ns/iter; 1.0583x vs baseline; 1.0583x over previous
import jax
import jax.numpy as jnp
from jax import lax
from jax.experimental import pallas as pl
from jax.experimental.pallas import tpu as pltpu

N_DEV = 4
M = 8192
N = 1024
HALF = M // 2
CHUNK = HALF // N_DEV
SUBS = 2
SUB = CHUNK // SUBS
HOPS = N_DEV - 1


def kernel(x):
    def body(x_ref, out_ref, stage, send0, rs_recv_a, rs_recv_b,
             dma_sems, rs_send_sems, rs_recv_sems, ag_send_sems, ag_recv_sems):
        my = lax.axis_index("i")
        left = lax.rem(my - 1 + N_DEV, N_DEV)
        right = lax.rem(my + 1, N_DEV)

        def rows(d, c, s):
            return pl.ds(d * HALF + c * CHUNK + s * SUB, SUB)

        def peer(d):
            return right if d == 0 else left

        def rs_chunk(d, h):
            if d == 0:
                return lax.rem(my - h - 1 + N_DEV, N_DEV)
            return lax.rem(my + h + 1, N_DEV)

        def ag_chunk(d, h):
            if d == 0:
                return lax.rem(my + 1 - h + N_DEV, N_DEV)
            return lax.rem(my + 3 + h, N_DEV)

        rs_recv = (rs_recv_a, rs_recv_b)

        def fetch(d, c, slot):
            return pltpu.make_async_copy(
                x_ref.at[pl.ds(d * HALF + c * CHUNK, CHUNK), :],
                stage.at[d, slot],
                dma_sems.at[d, slot],
            )

        def rs_rdma(d, h, s):
            if h == 0:
                src = send0.at[d, pl.ds(s * SUB, SUB), :]
            else:
                src = rs_recv[d].at[h - 1, pl.ds(s * SUB, SUB), :]
            return pltpu.make_async_remote_copy(
                src_ref=src,
                dst_ref=rs_recv[d].at[h, pl.ds(s * SUB, SUB), :],
                send_sem=rs_send_sems.at[d, h, s],
                recv_sem=rs_recv_sems.at[d, h, s],
                device_id=(peer(d),),
                device_id_type=pl.DeviceIdType.MESH,
            )

        def ag_rdma(d, h, s):
            r = rows(d, ag_chunk(d, h), s)
            return pltpu.make_async_remote_copy(
                src_ref=out_ref.at[r, :],
                dst_ref=out_ref.at[r, :],
                send_sem=ag_send_sems.at[d, h, s],
                recv_sem=ag_recv_sems.at[d, h, s],
                device_id=(peer(d),),
                device_id_type=pl.DeviceIdType.MESH,
            )

        for d in range(2):
            fetch(d, my, 0).start()
            fetch(d, rs_chunk(d, 0), 1).start()

        barrier_sem = pltpu.get_barrier_semaphore()
        for nbr in (left, right):
            pl.semaphore_signal(
                barrier_sem, inc=1,
                device_id=(nbr,), device_id_type=pl.DeviceIdType.MESH,
            )
        pl.semaphore_wait(barrier_sem, 2)

        for d in range(2):
            fetch(d, my, 0).wait()
            send0[d, :, :] = stage[d, 0].astype(jnp.bfloat16)
            for s in range(SUBS):
                rs_rdma(d, 0, s).start()
            fetch(d, rs_chunk(d, 1), 0).start()

        for h in range(1, HOPS):
            slot = h % 2
            for d in range(2):
                fetch(d, rs_chunk(d, h - 1), slot).wait()
            for s in range(SUBS):
                for d in range(2):
                    rs_rdma(d, h - 1, s).wait_recv()
                    sl = pl.ds(s * SUB, SUB)
                    rs_recv[d][h - 1, sl, :] = (
                        rs_recv[d][h - 1, sl, :]
                        + stage[d, slot, sl, :].astype(jnp.bfloat16)
                    )
                    rs_rdma(d, h, s).start()
            if h == 1:
                for d in range(2):
                    fetch(d, rs_chunk(d, 2), 1).start()

        for d in range(2):
            fetch(d, rs_chunk(d, HOPS - 1), 1).wait()
        for s in range(SUBS):
            for d in range(2):
                rs_rdma(d, HOPS - 1, s).wait_recv()
                c = rs_chunk(d, HOPS - 1)
                sl = pl.ds(s * SUB, SUB)
                out_ref[rows(d, c, s), :] = (
                    rs_recv[d][HOPS - 1, sl, :]
                    + stage[d, 1, sl, :].astype(jnp.bfloat16)
                )
                ag_rdma(d, 0, s).start()

        for h in range(1, HOPS):
            for s in range(SUBS):
                for d in range(2):
                    ag_rdma(d, h - 1, s).wait_recv()
                    ag_rdma(d, h, s).start()

        for s in range(SUBS):
            for d in range(2):
                ag_rdma(d, HOPS - 1, s).wait_recv()
        for h in range(HOPS):
            for s in range(SUBS):
                for d in range(2):
                    rs_rdma(d, h, s).wait_send()
                    ag_rdma(d, h, s).wait_send()

    return pl.pallas_call(
        body,
        out_shape=jax.ShapeDtypeStruct((M, N), jnp.bfloat16),
        in_specs=[pl.BlockSpec(memory_space=pl.ANY)],
        out_specs=pl.BlockSpec(memory_space=pltpu.VMEM),
        scratch_shapes=[
            pltpu.VMEM((2, 2, CHUNK, N), jnp.float32),
            pltpu.VMEM((2, CHUNK, N), jnp.bfloat16),
            pltpu.VMEM((HOPS, CHUNK, N), jnp.bfloat16),
            pltpu.VMEM((HOPS, CHUNK, N), jnp.bfloat16),
            pltpu.SemaphoreType.DMA((2, 2)),
            pltpu.SemaphoreType.DMA((2, HOPS, SUBS)),
            pltpu.SemaphoreType.DMA((2, HOPS, SUBS)),
            pltpu.SemaphoreType.DMA((2, HOPS, SUBS)),
            pltpu.SemaphoreType.DMA((2, HOPS, SUBS)),
        ],
        compiler_params=pltpu.CompilerParams(collective_id=0),
    )(x)
